# baseline (speedup 1.0000x reference)
import jax
import jax.numpy as jnp
from jax import lax
from jax.experimental import pallas as pl
from jax.experimental.pallas import tpu as pltpu
from jax.experimental.pallas import tpu_sc as plsc

BATCH = 16384
FEAT = 64
CLS = 1000000

NC = 2
NS = 16
NW = NC * NS
BW = BATCH // NW
ROWS2D = BATCH // 128
NGRP = BW // 16
RING = 2


def _sc_body(ys2d, center2, gath_out, cnt_out,
             idx_f, idx_g, idx_h, rows, cnt_v, src_v, hist, gsem):
    c = lax.axis_index("c")
    s = lax.axis_index("s")
    wid = s * NC + c

    pltpu.sync_copy(ys2d.at[pl.ds(wid * 4, 4)], idx_g)
    pltpu.sync_copy(ys2d.at[pl.ds(s * 8, 8)], idx_h)
    for j in range(4):
        for i in range(8):
            r = idx_g[j, pl.ds(i * 16, 16)]
            idx_f[pl.ds(j * 128 + i * 16, 16)] = lax.shift_right_logical(r, 1)

    for m in range(NGRP):
        v = idx_f[pl.ds(m * 16, 16)]
        for k in range(16):
            pltpu.async_copy(center2.at[pl.ds(v[k], 1)],
                             rows.at[pl.ds(m * 16 + k, 1)], gsem)

    for i in range(8):
        src_v[pl.ds(i * 16, 16)] = jnp.zeros((16,), jnp.float32)
    for j in range(8):
        pltpu.sync_copy(src_v, hist.at[idx_h.at[j]])
    plsc.subcore_barrier()
    for i in range(8):
        src_v[pl.ds(i * 16, 16)] = jnp.ones((16,), jnp.float32)
    for j in range(8):
        pltpu.sync_copy(src_v, hist.at[idx_h.at[j]], add=True)
    plsc.subcore_barrier()
    for j in range(4):
        pltpu.sync_copy(hist.at[idx_g.at[j]], cnt_v.at[j])
    for j in range(4):
        for i in range(8):
            r = idx_g[j, pl.ds(i * 16, 16)]
            sgn = (jnp.ones((16,), jnp.float32)
                   - 2.0 * lax.bitwise_and(
                       r, jnp.full((16,), 1, jnp.int32)).astype(jnp.float32))
            cnt_v[j, pl.ds(i * 16, 16)] = cnt_v[j, pl.ds(i * 16, 16)] * sgn
    pltpu.sync_copy(cnt_v, cnt_out.at[pl.ds(wid * 4, 4)])

    pltpu.make_async_copy(center2.at[pl.ds(0, BW)], rows, gsem).wait()
    pltpu.sync_copy(rows, gath_out.at[pl.ds(wid * BW, BW)])


def _sc_gather_count(ys2d, center2):
    mesh = plsc.VectorSubcoreMesh(core_axis_name="c", subcore_axis_name="s",
                                  num_cores=NC, num_subcores=NS)
    return pl.kernel(
        _sc_body,
        out_type=(
            jax.ShapeDtypeStruct((BATCH, 128), jnp.float32),
            jax.ShapeDtypeStruct((ROWS2D, 128), jnp.float32),
        ),
        mesh=mesh,
        scratch_types=[
            pltpu.VMEM((BW,), jnp.int32),
            pltpu.VMEM((4, 128), jnp.int32),
            pltpu.VMEM((8, 128), jnp.int32),
            pltpu.VMEM((BW, 128), jnp.float32),
            pltpu.VMEM((4, 128), jnp.float32),
            pltpu.VMEM((128,), jnp.float32),
            pltpu.VMEM_SHARED((CLS,), jnp.float32),
            pltpu.SemaphoreType.DMA,
        ],
    )(ys2d, center2)


def _tc_body(xs_ref, g_ref, cnt_ref, out_ref):
    xs = xs_ref[...]
    norm = jnp.maximum(jnp.sqrt(jnp.sum(xs * xs, axis=1, keepdims=True)),
                       1e-12)
    cnt_signed = cnt_ref[...]
    cnt = jnp.abs(cnt_signed)
    g = g_ref[...]
    c_sel = jnp.where(cnt_signed < 0, g[:, FEAT:], g[:, :FEAT])
    diff = xs / norm - c_sel
    dist = jnp.sqrt(jnp.sum(diff * diff, axis=1, keepdims=True))
    out_ref[...] = jnp.sum(dist / cnt).reshape(1, 1)


def _tc_loss(xs, gathered, cnt):
    return pl.pallas_call(
        _tc_body,
        out_shape=jax.ShapeDtypeStruct((1, 1), jnp.float32),
    )(xs, gathered, cnt)


def kernel(xs, ys, center):
    ys2d = ys.astype(jnp.int32).reshape(ROWS2D, 128)
    center2 = center.reshape(CLS // 2, 128)
    gathered, cnt2d = _sc_gather_count(ys2d, center2)
    out = _tc_loss(xs, gathered, cnt2d.reshape(BATCH, 1))
    return out.reshape(())


# speedup vs baseline: 1.6978x; 1.6978x over previous
import jax
import jax.numpy as jnp
from jax import lax
from jax.experimental import pallas as pl
from jax.experimental.pallas import tpu as pltpu
from jax.experimental.pallas import tpu_sc as plsc

BATCH = 16384
FEAT = 64
CLS = 1000000

NC = 2
NS = 16
NW = NC * NS
BW = BATCH // NW
ROWS2D = BATCH // 128
NGRP = BW // 16
RING = 2


def _sc_body(ys2d, center, gath_out, cnt_out,
             idx_f, idx_g, idx_h, rows, cnt_v, src_v, hist, gsem):
    c = lax.axis_index("c")
    s = lax.axis_index("s")
    wid = s * NC + c

    pltpu.sync_copy(ys2d.at[pl.ds(wid * 4, 4)], idx_g)
    pltpu.sync_copy(ys2d.at[pl.ds(s * 8, 8)], idx_h)
    for j in range(4):
        for i in range(8):
            idx_f[pl.ds(j * 128 + i * 16, 16)] = idx_g[j, pl.ds(i * 16, 16)]

    for m in range(NGRP):
        v = idx_f[pl.ds(m * 16, 16)]
        for k in range(16):
            pltpu.async_copy(center.at[pl.ds(v[k], 1)],
                             rows.at[pl.ds(m * 16 + k, 1)], gsem)

    for i in range(8):
        src_v[pl.ds(i * 16, 16)] = jnp.zeros((16,), jnp.float32)
    for j in range(8):
        pltpu.sync_copy(src_v, hist.at[idx_h.at[j]])
    plsc.subcore_barrier()
    for i in range(8):
        src_v[pl.ds(i * 16, 16)] = jnp.ones((16,), jnp.float32)
    for j in range(8):
        pltpu.sync_copy(src_v, hist.at[idx_h.at[j]], add=True)
    plsc.subcore_barrier()
    for j in range(4):
        pltpu.sync_copy(hist.at[idx_g.at[j]], cnt_v.at[j])
    pltpu.sync_copy(cnt_v, cnt_out.at[pl.ds(wid * 4, 4)])

    pltpu.make_async_copy(center.at[pl.ds(0, BW)], rows, gsem).wait()
    pltpu.sync_copy(rows, gath_out.at[pl.ds(wid * BW, BW)])


def _sc_gather_count(ys2d, center):
    mesh = plsc.VectorSubcoreMesh(core_axis_name="c", subcore_axis_name="s",
                                  num_cores=NC, num_subcores=NS)
    return pl.kernel(
        _sc_body,
        out_type=(
            jax.ShapeDtypeStruct((BATCH, FEAT), jnp.float32),
            jax.ShapeDtypeStruct((ROWS2D, 128), jnp.float32),
        ),
        mesh=mesh,
        scratch_types=[
            pltpu.VMEM((BW,), jnp.int32),
            pltpu.VMEM((4, 128), jnp.int32),
            pltpu.VMEM((8, 128), jnp.int32),
            pltpu.VMEM((BW, FEAT), jnp.float32),
            pltpu.VMEM((4, 128), jnp.float32),
            pltpu.VMEM((128,), jnp.float32),
            pltpu.VMEM_SHARED((CLS,), jnp.float32),
            pltpu.SemaphoreType.DMA,
        ],
    )(ys2d, center)


def _tc_body(xs_ref, g_ref, cnt_ref, out_ref):
    xs = xs_ref[...]
    norm = jnp.maximum(jnp.sqrt(jnp.sum(xs * xs, axis=1, keepdims=True)),
                       1e-12)
    diff = xs / norm - g_ref[...]
    dist = jnp.sqrt(jnp.sum(diff * diff, axis=1, keepdims=True))
    out_ref[...] = jnp.sum(dist / cnt_ref[...]).reshape(1, 1)


def _tc_loss(xs, gathered, cnt):
    return pl.pallas_call(
        _tc_body,
        out_shape=jax.ShapeDtypeStruct((1, 1), jnp.float32),
    )(xs, gathered, cnt)


def kernel(xs, ys, center):
    ys2d = ys.astype(jnp.int32).reshape(ROWS2D, 128)
    gathered, cnt2d = _sc_gather_count(ys2d, center)
    out = _tc_loss(xs, gathered, cnt2d.reshape(BATCH, 1))
    return out.reshape(())
